# Initial kernel scaffold; baseline (speedup 1.0000x reference)
import jax
import jax.numpy as jnp
from jax.experimental import pallas as pl


def kernel(x, edge_index, W1, b1, We):
    raise NotImplementedError("write your pallas kernel here")


# speedup vs baseline: 15.9353x; 15.9353x over previous
import functools

import jax
import jax.numpy as jnp
from jax import lax
from jax.experimental import pallas as pl
from jax.experimental.pallas import tpu as pltpu
from jax.experimental.pallas import tpu_sc as plsc

N = 10000
NPAD = 10240
E = 320000
EPAD = 327680
F = 128
H = 64

NSUB = 16
ROWS_PT = NPAD // NSUB
EPT = EPAD // NSUB
BPT = EPT // 128
GRP = 8
NGRP = BPT // GRP

_MESH = plsc.VectorSubcoreMesh(core_axis_name="c", subcore_axis_name="s",
                               num_cores=2, num_subcores=NSUB)


def _zero_rows(buf, n_rows, width):
    zv = jnp.zeros((16,), jnp.float32)

    def body(r, carry):
        for j in range(width // 16):
            buf[r, pl.ds(j * 16, 16)] = zv
        return carry

    lax.fori_loop(0, n_rows, body, 0)


def _make_wide_pass(relu):
    @functools.partial(
        pl.kernel,
        out_type=jax.ShapeDtypeStruct((2 * NPAD, 32), jnp.float32),
        mesh=_MESH,
        compiler_params=pltpu.CompilerParams(use_tc_tiling_on_sc=False),
        scratch_types=[
            pltpu.VMEM((GRP, 128), jnp.int32),
            pltpu.VMEM((GRP, 128), jnp.int32),
            pltpu.VMEM((GRP * 128, 32), jnp.float32),
            pltpu.VMEM((ROWS_PT, 32), jnp.float32),
            pltpu.VMEM((ROWS_PT, 32), jnp.float32),
            pltpu.VMEM((ROWS_PT,), jnp.float32),
            pltpu.VMEM((32,), jnp.float32),
            pltpu.VMEM_SHARED((NPAD, 32), jnp.float32),
            pltpu.SemaphoreType.DMA,
            pltpu.SemaphoreType.DMA,
        ],
    )
    def kern(cin, srcb2, dst2, addend, scale, bias, out,
             src_v, dst_v, rows_v, wb_v, a_v, s_v, b_v, acc, gsem, ssem):
        c = lax.axis_index("c")
        s = lax.axis_index("s")
        row0 = s * ROWS_PT

        _zero_rows(wb_v, ROWS_PT, 32)
        pltpu.sync_copy(wb_v, acc.at[pl.ds(row0, ROWS_PT)])
        plsc.subcore_barrier()

        src_row0 = c * (2 * EPAD // 128 // 2) + s * BPT
        dst_row0 = s * BPT

        def grp_body(g, carry):
            pltpu.sync_copy(srcb2.at[pl.ds(src_row0 + g * GRP, GRP)], src_v)
            pltpu.sync_copy(dst2.at[pl.ds(dst_row0 + g * GRP, GRP)], dst_v)
            gathers = [
                pltpu.async_copy(
                    cin.at[src_v.at[j]],
                    rows_v.at[pl.ds(j * 128, 128)],
                    gsem,
                )
                for j in range(GRP)
            ]
            for cp in gathers:
                cp.wait()
            scatters = [
                pltpu.async_copy(
                    rows_v.at[pl.ds(j * 128, 128)],
                    acc.at[dst_v.at[j]],
                    ssem,
                    add=True,
                )
                for j in range(GRP)
            ]
            for cp in scatters:
                cp.wait()
            return carry

        lax.fori_loop(0, NGRP, grp_body, 0)
        plsc.subcore_barrier()

        pltpu.sync_copy(acc.at[pl.ds(row0, ROWS_PT)], wb_v)
        pltpu.sync_copy(addend.at[pl.ds(c * NPAD + row0, ROWS_PT)], a_v)
        pltpu.sync_copy(scale.at[pl.ds(row0, ROWS_PT)], s_v)
        pltpu.sync_copy(bias.at[pl.ds(c * 32, 32)], b_v)

        def wb_body(i, carry):
            svals = s_v[pl.ds(i * 16, 16)]
            for rr in range(16):
                r = i * 16 + rr
                sval = svals[rr]
                for j in range(2):
                    sl = pl.ds(j * 16, 16)
                    v = a_v[r, sl] + sval * wb_v[r, sl] + b_v[sl]
                    if relu:
                        v = jnp.maximum(v, 0.0)
                    wb_v[r, sl] = v
            return carry

        lax.fori_loop(0, ROWS_PT // 16, wb_body, 0)
        pltpu.sync_copy(wb_v, out.at[pl.ds(c * NPAD + row0, ROWS_PT)])

    return kern


_wide_plain = _make_wide_pass(False)
_wide_relu = _make_wide_pass(True)


@functools.partial(
    pl.kernel,
    out_type=jax.ShapeDtypeStruct((NPAD,), jnp.float32),
    mesh=_MESH,
    compiler_params=pltpu.CompilerParams(use_tc_tiling_on_sc=False),
    scratch_types=[
        pltpu.VMEM((GRP, 128), jnp.int32),
        pltpu.VMEM((GRP, 128), jnp.int32),
        pltpu.VMEM((GRP * 128,), jnp.float32),
        pltpu.VMEM((ROWS_PT,), jnp.float32),
        pltpu.VMEM((ROWS_PT,), jnp.float32),
        pltpu.VMEM((ROWS_PT,), jnp.float32),
        pltpu.VMEM_SHARED((NPAD,), jnp.float32),
        pltpu.SemaphoreType.DMA,
        pltpu.SemaphoreType.DMA,
    ],
)
def _scalar_pass(cin, src2, dst2, addend, scale, out,
                 src_v, dst_v, vals_v, wb_v, a_v, s_v, acc, gsem, ssem):
    c = lax.axis_index("c")
    s = lax.axis_index("s")

    @pl.when(c == 0)
    def _():
        row0 = s * ROWS_PT
        zv = jnp.zeros((16,), jnp.float32)

        def zb(i, carry):
            wb_v[pl.ds(i * 16, 16)] = zv
            return carry

        lax.fori_loop(0, ROWS_PT // 16, zb, 0)
        pltpu.sync_copy(wb_v, acc.at[pl.ds(row0, ROWS_PT)])
        plsc.subcore_barrier()

        def grp_body(g, carry):
            rb = s * BPT + g * GRP
            pltpu.sync_copy(src2.at[pl.ds(rb, GRP)], src_v)
            pltpu.sync_copy(dst2.at[pl.ds(rb, GRP)], dst_v)
            gathers = [
                pltpu.async_copy(
                    cin.at[src_v.at[j]],
                    vals_v.at[pl.ds(j * 128, 128)],
                    gsem,
                )
                for j in range(GRP)
            ]
            for cp in gathers:
                cp.wait()
            scatters = [
                pltpu.async_copy(
                    vals_v.at[pl.ds(j * 128, 128)],
                    acc.at[dst_v.at[j]],
                    ssem,
                    add=True,
                )
                for j in range(GRP)
            ]
            for cp in scatters:
                cp.wait()
            return carry

        lax.fori_loop(0, NGRP, grp_body, 0)
        plsc.subcore_barrier()

        pltpu.sync_copy(acc.at[pl.ds(row0, ROWS_PT)], wb_v)
        pltpu.sync_copy(addend.at[pl.ds(row0, ROWS_PT)], a_v)
        pltpu.sync_copy(scale.at[pl.ds(row0, ROWS_PT)], s_v)

        def wb_body(i, carry):
            sl = pl.ds(i * 16, 16)
            wb_v[sl] = a_v[sl] + s_v[sl] * wb_v[sl]
            return carry

        lax.fori_loop(0, ROWS_PT // 16, wb_body, 0)
        pltpu.sync_copy(wb_v, out.at[pl.ds(row0, ROWS_PT)])


_BLK_N = 256
_NBLK_N = NPAD // _BLK_N


def _tc_prep(x_pad, w1r, deg2):
    def body(x_ref, w_ref, deg_ref, c1_ref, a2_ref, a1_ref, a0_ref,
             dinv_ref, dgi_ref):
        y = jnp.dot(x_ref[...], w_ref[...],
                    preferred_element_type=jnp.float32)
        deg = deg_ref[...]
        dinv = jnp.where(deg > 0.0, lax.rsqrt(deg), 0.0)
        dinv_ref[...] = dinv
        dgi_ref[...] = dinv * dinv

        def split(z):
            return jnp.stack([z[:, :32], z[:, 32:]], axis=0)

        c1_ref[...] = split(dinv * y[:, 192:256])
        a2_ref[...] = split(dinv * y[:, 128:192])
        a1_ref[...] = split(dinv * y[:, 64:128])
        a0_ref[...] = split(y[:, 0:64])

    split_shape = jax.ShapeDtypeStruct((2, NPAD, 32), jnp.float32)
    col_shape = jax.ShapeDtypeStruct((NPAD, 1), jnp.float32)
    split_spec = pl.BlockSpec((2, _BLK_N, 32), lambda i: (0, i, 0))
    col_spec = pl.BlockSpec((_BLK_N, 1), lambda i: (i, 0))
    return pl.pallas_call(
        body,
        grid=(_NBLK_N,),
        in_specs=[
            pl.BlockSpec((_BLK_N, F), lambda i: (i, 0)),
            pl.BlockSpec((F, 4 * H), lambda i: (0, 0)),
            col_spec,
        ],
        out_specs=[split_spec, split_spec, split_spec, split_spec,
                   col_spec, col_spec],
        out_shape=[split_shape, split_shape, split_shape, split_shape,
                   col_shape, col_shape],
    )(x_pad, w1r, deg2)


def _tc_post(h_split, wq, dinv2):
    def body(h_ref, wq_ref, dinv_ref, p3_ref, p2_ref, p1_ref, p0_ref):
        i = pl.program_id(0)
        hs = h_ref[...]
        hb = jnp.concatenate([hs[0], hs[1]], axis=1)
        q = jnp.dot(hb, wq_ref[...], preferred_element_type=jnp.float32)
        rows = i * _BLK_N + lax.broadcasted_iota(jnp.int32, (_BLK_N, 1), 0)
        q = jnp.where(rows < N, q, 0.0)
        dinv = dinv_ref[...]
        p3_ref[...] = dinv * q[:, 3:4]
        p2_ref[...] = dinv * q[:, 2:3]
        p1_ref[...] = dinv * q[:, 1:2]
        p0_ref[...] = q[:, 0:1]

    col_shape = jax.ShapeDtypeStruct((NPAD, 1), jnp.float32)
    col_spec = pl.BlockSpec((_BLK_N, 1), lambda i: (i, 0))
    return pl.pallas_call(
        body,
        grid=(_NBLK_N,),
        in_specs=[
            pl.BlockSpec((2, _BLK_N, 32), lambda i: (0, i, 0)),
            pl.BlockSpec((H, 4), lambda i: (0, 0)),
            col_spec,
        ],
        out_specs=[col_spec, col_spec, col_spec, col_spec],
        out_shape=[col_shape, col_shape, col_shape, col_shape],
    )(h_split, wq, dinv2)


def _tc_final(svec2):
    def body(s_ref, out_ref):
        total = jnp.sum(s_ref[...])
        out_ref[...] = jax.nn.sigmoid(total / N).reshape(1, 1)

    return pl.pallas_call(
        body,
        grid=(1,),
        in_specs=[pl.BlockSpec((NPAD, 1), lambda i: (0, 0))],
        out_specs=pl.BlockSpec((1, 1), lambda i: (0, 0)),
        out_shape=jax.ShapeDtypeStruct((1, 1), jnp.float32),
    )(svec2)


def kernel(x, edge_index, W1, b1, We):
    src = edge_index[0].astype(jnp.int32)
    dst = edge_index[1].astype(jnp.int32)

    npd = EPAD - E
    pad_rows = N + (jnp.arange(npd, dtype=jnp.int32) % (NPAD - N))
    src_all = jnp.concatenate([src, pad_rows])
    dst_all = jnp.concatenate([dst, pad_rows])
    srcb2 = jnp.concatenate([src_all, src_all + NPAD]).reshape(-1, 128)
    srcs2 = src_all.reshape(-1, 128)
    dst2 = dst_all.reshape(-1, 128)

    x_pad = jnp.pad(x, ((0, NPAD - N), (0, 0)))
    w1r = W1.reshape(4, F, H).transpose(1, 0, 2).reshape(F, 4 * H)
    wq = We.reshape(4, H).T

    ones_n = jnp.where(jnp.arange(NPAD) < N, 1.0, 0.0).astype(jnp.float32)
    zeros_np = jnp.zeros((NPAD,), jnp.float32)
    ones_np = jnp.ones((NPAD,), jnp.float32)
    zeros64 = jnp.zeros((64,), jnp.float32)

    deg = _scalar_pass(ones_n, srcs2, dst2, zeros_np, ones_np)

    c1, a2, a1, a0, dinv2, dgi2 = _tc_prep(x_pad, w1r, deg.reshape(NPAD, 1))
    dinv = dinv2.reshape(NPAD)
    dgi = dgi2.reshape(NPAD)
    r2 = lambda z: z.reshape(2 * NPAD, 32)

    t = _wide_plain(r2(c1), srcb2, dst2, r2(a2), dgi, zeros64)
    t = _wide_plain(t, srcb2, dst2, r2(a1), dgi, zeros64)
    h = _wide_relu(t, srcb2, dst2, r2(a0), dinv, b1)

    p3, p2, p1, p0 = _tc_post(h.reshape(2, NPAD, 32), wq, dinv2)

    t = _scalar_pass(p3.reshape(NPAD), srcs2, dst2, p2.reshape(NPAD), dgi)
    t = _scalar_pass(t, srcs2, dst2, p1.reshape(NPAD), dgi)
    svec = _scalar_pass(t, srcs2, dst2, p0.reshape(NPAD), dinv)

    return _tc_final(svec.reshape(NPAD, 1))
